# Initial kernel scaffold; baseline (speedup 1.0000x reference)
import jax
import jax.numpy as jnp
from jax.experimental import pallas as pl


def kernel(x, edge_index, edge_weight, edge_color, W1, root1, b1, W2, root2, b2):
    raise NotImplementedError("write your pallas kernel here")


# speedup vs baseline: 9.2563x; 9.2563x over previous
import functools

import jax
import jax.numpy as jnp
from jax import lax
from jax.experimental import pallas as pl
from jax.experimental.pallas import tpu as pltpu
from jax.experimental.pallas import tpu_sc as plsc

N = 10000
E = 320000
F_IN = 128
HID = 16
NCLS = 7
R = 10

NC = 2
NS = 16
NW = NC * NS
CHUNK = 128
CHUNKS_PER_W = 80
E_PAD = NW * CHUNKS_PER_W * CHUNK
N_ACC = 10240
ROWS_PER_TILE = N_ACC // NS
BN = 2000
GRID_N = N // BN


def _edge_pass(table, gidx, dst, wt):
    mesh = plsc.VectorSubcoreMesh(
        core_axis_name="c", subcore_axis_name="s", num_cores=NC, num_subcores=NS
    )

    @functools.partial(
        pl.kernel,
        out_type=jax.ShapeDtypeStruct((NC, N_ACC, HID), jnp.float32),
        mesh=mesh,
        compiler_params=pltpu.CompilerParams(use_tc_tiling_on_sc=False),
        scratch_types=[
            pltpu.VMEM((CHUNKS_PER_W, CHUNK), jnp.int32),
            pltpu.VMEM((CHUNKS_PER_W, CHUNK), jnp.int32),
            pltpu.VMEM((CHUNKS_PER_W, CHUNK), jnp.float32),
            pltpu.VMEM((CHUNK, HID), jnp.float32),
            pltpu.VMEM((ROWS_PER_TILE, HID), jnp.float32),
            pltpu.VMEM_SHARED((N_ACC, HID), jnp.float32),
        ],
    )
    def k(table_hbm, gidx_hbm, dst_hbm, wt_hbm, out_hbm,
          gidx_v, dst_v, wt_v, rows_v, zbuf_v, acc_sh):
        c = lax.axis_index("c")
        s = lax.axis_index("s")
        w = c * NS + s

        def zrow(i, carry):
            zbuf_v[i] = jnp.zeros((HID,), jnp.float32)
            return carry

        lax.fori_loop(0, ROWS_PER_TILE, zrow, 0)
        pltpu.sync_copy(
            zbuf_v, acc_sh.at[pl.ds(s * ROWS_PER_TILE, ROWS_PER_TILE)]
        )

        pltpu.sync_copy(gidx_hbm.at[w], gidx_v)
        pltpu.sync_copy(dst_hbm.at[w], dst_v)
        pltpu.sync_copy(wt_hbm.at[w], wt_v)
        plsc.subcore_barrier()

        def chunk(j, carry):
            pltpu.sync_copy(table_hbm.at[gidx_v.at[j]], rows_v)

            def mul16(g, c2):
                wv = wt_v[j, pl.ds(g * HID, HID)]
                base = g * HID
                for kk in range(HID):
                    rows_v[base + kk] = rows_v[base + kk] * wv[kk]
                return c2

            lax.fori_loop(0, CHUNK // HID, mul16, 0)
            pltpu.sync_copy(rows_v, acc_sh.at[dst_v.at[j]], add=True)
            return carry

        lax.fori_loop(0, CHUNKS_PER_W, chunk, 0)
        plsc.subcore_barrier()
        pltpu.sync_copy(
            acc_sh.at[pl.ds(s * ROWS_PER_TILE, ROWS_PER_TILE)],
            out_hbm.at[c, pl.ds(s * ROWS_PER_TILE, ROWS_PER_TILE)],
        )

    return k(table, gidx, dst, wt)


def _tc_transform1(x, wrel, wroot):
    def body(x_ref, w_ref, r_ref, rel_ref, root_ref):
        xb = x_ref[...]
        rel_ref[...] = jnp.dot(xb, w_ref[...], preferred_element_type=jnp.float32)
        root_ref[...] = jnp.dot(xb, r_ref[...], preferred_element_type=jnp.float32)

    return pl.pallas_call(
        body,
        grid=(GRID_N,),
        in_specs=[
            pl.BlockSpec((BN, F_IN), lambda i: (i, 0)),
            pl.BlockSpec((F_IN, R * HID), lambda i: (0, 0)),
            pl.BlockSpec((F_IN, HID), lambda i: (0, 0)),
        ],
        out_specs=[
            pl.BlockSpec((BN, R * HID), lambda i: (i, 0)),
            pl.BlockSpec((BN, HID), lambda i: (i, 0)),
        ],
        out_shape=[
            jax.ShapeDtypeStruct((N, R * HID), jnp.float32),
            jax.ShapeDtypeStruct((N, HID), jnp.float32),
        ],
    )(x, wrel, wroot)


def _tc_mid(agg1, troot, b1, wrel2, root2p):
    def body(agg_ref, troot_ref, b1_ref, w_ref, r_ref, rel_ref, hroot_ref):
        h = agg_ref[0] + agg_ref[1] + troot_ref[...] + b1_ref[...]
        h = jnp.maximum(h, 0.0)
        rel_ref[...] = jnp.dot(h, w_ref[...], preferred_element_type=jnp.float32)
        hroot_ref[...] = jnp.dot(h, r_ref[...], preferred_element_type=jnp.float32)

    return pl.pallas_call(
        body,
        grid=(GRID_N,),
        in_specs=[
            pl.BlockSpec((NC, BN, HID), lambda i: (0, i, 0)),
            pl.BlockSpec((BN, HID), lambda i: (i, 0)),
            pl.BlockSpec((1, HID), lambda i: (0, 0)),
            pl.BlockSpec((HID, R * HID), lambda i: (0, 0)),
            pl.BlockSpec((HID, HID), lambda i: (0, 0)),
        ],
        out_specs=[
            pl.BlockSpec((BN, R * HID), lambda i: (i, 0)),
            pl.BlockSpec((BN, HID), lambda i: (i, 0)),
        ],
        out_shape=[
            jax.ShapeDtypeStruct((N, R * HID), jnp.float32),
            jax.ShapeDtypeStruct((N, HID), jnp.float32),
        ],
    )(agg1, troot, b1, wrel2, root2p)


def _tc_final(agg2, hroot, b2p):
    def body(agg_ref, hroot_ref, b2_ref, ls_ref, o_ref):
        o = agg_ref[0] + agg_ref[1] + hroot_ref[...] + b2_ref[...]
        col = lax.broadcasted_iota(jnp.int32, (BN, HID), 1)
        mask = col < NCLS
        om = jnp.where(mask, o, jnp.float32(-1e30))
        m = jnp.max(om, axis=1, keepdims=True)
        e = jnp.where(mask, jnp.exp(o - m), 0.0)
        ssum = jnp.sum(e, axis=1, keepdims=True)
        ls_ref[...] = o - m - jnp.log(ssum)
        o_ref[...] = o

    return pl.pallas_call(
        body,
        grid=(GRID_N,),
        in_specs=[
            pl.BlockSpec((NC, BN, HID), lambda i: (0, i, 0)),
            pl.BlockSpec((BN, HID), lambda i: (i, 0)),
            pl.BlockSpec((1, HID), lambda i: (0, 0)),
        ],
        out_specs=[
            pl.BlockSpec((BN, HID), lambda i: (i, 0)),
            pl.BlockSpec((BN, HID), lambda i: (i, 0)),
        ],
        out_shape=[
            jax.ShapeDtypeStruct((N, HID), jnp.float32),
            jax.ShapeDtypeStruct((N, HID), jnp.float32),
        ],
    )(agg2, hroot, b2p)


def kernel(x, edge_index, edge_weight, edge_color, W1, root1, b1, W2, root2, b2):
    src = edge_index[0].astype(jnp.int32)
    dst = edge_index[1].astype(jnp.int32)
    col = edge_color.astype(jnp.int32)
    gidx = src * R + col
    pad = E_PAD - E
    gidx = jnp.pad(gidx, (0, pad)).reshape(NW, CHUNKS_PER_W, CHUNK)
    dstp = jnp.pad(dst, (0, pad)).reshape(NW, CHUNKS_PER_W, CHUNK)
    wtp = jnp.pad(edge_weight, (0, pad)).reshape(NW, CHUNKS_PER_W, CHUNK)

    wrel1 = W1.transpose(1, 0, 2).reshape(F_IN, R * HID)
    t_rel1, t_root1 = _tc_transform1(x, wrel1, root1)
    agg1 = _edge_pass(t_rel1.reshape(N * R, HID), gidx, dstp, wtp)

    wrel2 = jnp.pad(W2, ((0, 0), (0, 0), (0, HID - NCLS)))
    wrel2 = wrel2.transpose(1, 0, 2).reshape(HID, R * HID)
    root2p = jnp.pad(root2, ((0, 0), (0, HID - NCLS)))
    t_rel2, t_hroot = _tc_mid(agg1, t_root1, b1.reshape(1, HID), wrel2, root2p)
    agg2 = _edge_pass(t_rel2.reshape(N * R, HID), gidx, dstp, wtp)

    b2p = jnp.pad(b2, (0, HID - NCLS)).reshape(1, HID)
    ls, o = _tc_final(agg2, t_hroot, b2p)
    return (ls[:, :NCLS], o[:, :NCLS])


# speedup vs baseline: 10.2357x; 1.1058x over previous
import functools

import jax
import jax.numpy as jnp
from jax import lax
from jax.experimental import pallas as pl
from jax.experimental.pallas import tpu as pltpu
from jax.experimental.pallas import tpu_sc as plsc

N = 10000
E = 320000
F_IN = 128
HID = 16
NCLS = 7
R = 10

NC = 2
NS = 16
NW = NC * NS
CHUNK = 128
CHUNKS_PER_W = 81
E_PAD = NW * CHUNKS_PER_W * CHUNK
N_ACC = 10240
ROWS_PER_TILE = N_ACC // NS
BN = 2000
GRID_N = N // BN


def _edge_pass(table, gidx, dst, wt):
    mesh = plsc.VectorSubcoreMesh(
        core_axis_name="c", subcore_axis_name="s", num_cores=NC, num_subcores=NS
    )

    @functools.partial(
        pl.kernel,
        out_type=jax.ShapeDtypeStruct((NC, N_ACC, HID), jnp.float32),
        mesh=mesh,
        compiler_params=pltpu.CompilerParams(use_tc_tiling_on_sc=False),
        scratch_types=[
            pltpu.VMEM((CHUNKS_PER_W, CHUNK), jnp.int32),
            pltpu.VMEM((CHUNKS_PER_W, CHUNK), jnp.int32),
            pltpu.VMEM((CHUNKS_PER_W, CHUNK), jnp.float32),
            pltpu.VMEM((CHUNK, HID), jnp.float32),
            pltpu.VMEM((CHUNK, HID), jnp.float32),
            pltpu.VMEM((CHUNK, HID), jnp.float32),
            pltpu.VMEM((ROWS_PER_TILE, HID), jnp.float32),
            pltpu.VMEM_SHARED((N_ACC, HID), jnp.float32),
            pltpu.SemaphoreType.DMA,
            pltpu.SemaphoreType.DMA,
            pltpu.SemaphoreType.DMA,
            pltpu.SemaphoreType.DMA,
            pltpu.SemaphoreType.DMA,
            pltpu.SemaphoreType.DMA,
        ],
    )
    def k(table_hbm, gidx_hbm, dst_hbm, wt_hbm, out_hbm,
          gidx_v, dst_v, wt_v, rows0_v, rows1_v, rows2_v, zbuf_v, acc_sh,
          gsem0, gsem1, gsem2, ssem0, ssem1, ssem2):
        rows = (rows0_v, rows1_v, rows2_v)
        gsem = (gsem0, gsem1, gsem2)
        ssem = (ssem0, ssem1, ssem2)
        c = lax.axis_index("c")
        s = lax.axis_index("s")
        w = c * NS + s

        def zrow(i, carry):
            zbuf_v[i] = jnp.zeros((HID,), jnp.float32)
            return carry

        lax.fori_loop(0, ROWS_PER_TILE, zrow, 0)
        pltpu.sync_copy(
            zbuf_v, acc_sh.at[pl.ds(s * ROWS_PER_TILE, ROWS_PER_TILE)]
        )

        pltpu.sync_copy(gidx_hbm.at[w], gidx_v)
        pltpu.sync_copy(dst_hbm.at[w], dst_v)
        pltpu.sync_copy(wt_hbm.at[w], wt_v)
        plsc.subcore_barrier()

        def mul_chunk(j, buf):
            def mul16(g, c2):
                wv = wt_v[j, pl.ds(g * HID, HID)]
                base = g * HID
                for kk in range(HID):
                    buf[base + kk] = buf[base + kk] * wv[kk]
                return c2

            lax.fori_loop(0, CHUNK // HID, mul16, 0)

        pltpu.async_copy(table_hbm.at[gidx_v.at[0]], rows[0], gsem[0])
        pltpu.async_copy(zbuf_v.at[pl.ds(0, CHUNK)], acc_sh.at[dst_v.at[0]],
                         ssem[1], add=True)
        pltpu.async_copy(zbuf_v.at[pl.ds(0, CHUNK)], acc_sh.at[dst_v.at[0]],
                         ssem[2], add=True)

        def triple(p, carry):
            for b in (0, 1, 2):
                j = 3 * p + b
                b_free = (b + 1) % 3
                jn = jnp.minimum(j + 1, CHUNKS_PER_W - 1)
                pltpu.make_async_copy(
                    rows[b_free], acc_sh.at[dst_v.at[j]], ssem[b_free]
                ).wait()
                pltpu.async_copy(table_hbm.at[gidx_v.at[jn]], rows[b_free],
                                 gsem[b_free])
                pltpu.make_async_copy(
                    table_hbm.at[gidx_v.at[j]], rows[b], gsem[b]
                ).wait()
                mul_chunk(j, rows[b])
                pltpu.async_copy(rows[b], acc_sh.at[dst_v.at[j]], ssem[b],
                                 add=True)
            return carry

        lax.fori_loop(0, CHUNKS_PER_W // 3, triple, 0)
        jl = CHUNKS_PER_W - 1
        pltpu.make_async_copy(table_hbm.at[gidx_v.at[jl]], rows[0], gsem[0]).wait()
        pltpu.make_async_copy(rows[1], acc_sh.at[dst_v.at[jl]], ssem[1]).wait()
        pltpu.make_async_copy(rows[2], acc_sh.at[dst_v.at[jl]], ssem[2]).wait()
        plsc.subcore_barrier()
        pltpu.sync_copy(
            acc_sh.at[pl.ds(s * ROWS_PER_TILE, ROWS_PER_TILE)],
            out_hbm.at[c, pl.ds(s * ROWS_PER_TILE, ROWS_PER_TILE)],
        )

    return k(table, gidx, dst, wt)


def _tc_transform1(x, wrel, wroot):
    def body(x_ref, w_ref, r_ref, rel_ref, root_ref):
        xb = x_ref[...]
        rel_ref[...] = jnp.dot(xb, w_ref[...], preferred_element_type=jnp.float32)
        root_ref[...] = jnp.dot(xb, r_ref[...], preferred_element_type=jnp.float32)

    return pl.pallas_call(
        body,
        grid=(GRID_N,),
        in_specs=[
            pl.BlockSpec((BN, F_IN), lambda i: (i, 0)),
            pl.BlockSpec((F_IN, R * HID), lambda i: (0, 0)),
            pl.BlockSpec((F_IN, HID), lambda i: (0, 0)),
        ],
        out_specs=[
            pl.BlockSpec((BN, R * HID), lambda i: (i, 0)),
            pl.BlockSpec((BN, HID), lambda i: (i, 0)),
        ],
        out_shape=[
            jax.ShapeDtypeStruct((N, R * HID), jnp.float32),
            jax.ShapeDtypeStruct((N, HID), jnp.float32),
        ],
    )(x, wrel, wroot)


def _tc_mid(agg1, troot, b1, wrel2, root2p):
    def body(agg_ref, troot_ref, b1_ref, w_ref, r_ref, rel_ref, hroot_ref):
        h = agg_ref[0] + agg_ref[1] + troot_ref[...] + b1_ref[...]
        h = jnp.maximum(h, 0.0)
        rel_ref[...] = jnp.dot(h, w_ref[...], preferred_element_type=jnp.float32)
        hroot_ref[...] = jnp.dot(h, r_ref[...], preferred_element_type=jnp.float32)

    return pl.pallas_call(
        body,
        grid=(GRID_N,),
        in_specs=[
            pl.BlockSpec((NC, BN, HID), lambda i: (0, i, 0)),
            pl.BlockSpec((BN, HID), lambda i: (i, 0)),
            pl.BlockSpec((1, HID), lambda i: (0, 0)),
            pl.BlockSpec((HID, R * HID), lambda i: (0, 0)),
            pl.BlockSpec((HID, HID), lambda i: (0, 0)),
        ],
        out_specs=[
            pl.BlockSpec((BN, R * HID), lambda i: (i, 0)),
            pl.BlockSpec((BN, HID), lambda i: (i, 0)),
        ],
        out_shape=[
            jax.ShapeDtypeStruct((N, R * HID), jnp.float32),
            jax.ShapeDtypeStruct((N, HID), jnp.float32),
        ],
    )(agg1, troot, b1, wrel2, root2p)


def _tc_final(agg2, hroot, b2p):
    def body(agg_ref, hroot_ref, b2_ref, ls_ref, o_ref):
        o = agg_ref[0] + agg_ref[1] + hroot_ref[...] + b2_ref[...]
        col = lax.broadcasted_iota(jnp.int32, (BN, HID), 1)
        mask = col < NCLS
        om = jnp.where(mask, o, jnp.float32(-1e30))
        m = jnp.max(om, axis=1, keepdims=True)
        e = jnp.where(mask, jnp.exp(o - m), 0.0)
        ssum = jnp.sum(e, axis=1, keepdims=True)
        ls_ref[...] = o - m - jnp.log(ssum)
        o_ref[...] = o

    return pl.pallas_call(
        body,
        grid=(GRID_N,),
        in_specs=[
            pl.BlockSpec((NC, BN, HID), lambda i: (0, i, 0)),
            pl.BlockSpec((BN, HID), lambda i: (i, 0)),
            pl.BlockSpec((1, HID), lambda i: (0, 0)),
        ],
        out_specs=[
            pl.BlockSpec((BN, HID), lambda i: (i, 0)),
            pl.BlockSpec((BN, HID), lambda i: (i, 0)),
        ],
        out_shape=[
            jax.ShapeDtypeStruct((N, HID), jnp.float32),
            jax.ShapeDtypeStruct((N, HID), jnp.float32),
        ],
    )(agg2, hroot, b2p)


def kernel(x, edge_index, edge_weight, edge_color, W1, root1, b1, W2, root2, b2):
    src = edge_index[0].astype(jnp.int32)
    dst = edge_index[1].astype(jnp.int32)
    col = edge_color.astype(jnp.int32)
    gidx = src * R + col
    pad = E_PAD - E
    gidx = jnp.pad(gidx, (0, pad)).reshape(NW, CHUNKS_PER_W, CHUNK)
    dstp = jnp.pad(dst, (0, pad)).reshape(NW, CHUNKS_PER_W, CHUNK)
    wtp = jnp.pad(edge_weight, (0, pad)).reshape(NW, CHUNKS_PER_W, CHUNK)

    wrel1 = W1.transpose(1, 0, 2).reshape(F_IN, R * HID)
    t_rel1, t_root1 = _tc_transform1(x, wrel1, root1)
    agg1 = _edge_pass(t_rel1.reshape(N * R, HID), gidx, dstp, wtp)

    wrel2 = jnp.pad(W2, ((0, 0), (0, 0), (0, HID - NCLS)))
    wrel2 = wrel2.transpose(1, 0, 2).reshape(HID, R * HID)
    root2p = jnp.pad(root2, ((0, 0), (0, HID - NCLS)))
    t_rel2, t_hroot = _tc_mid(agg1, t_root1, b1.reshape(1, HID), wrel2, root2p)
    agg2 = _edge_pass(t_rel2.reshape(N * R, HID), gidx, dstp, wtp)

    b2p = jnp.pad(b2, (0, HID - NCLS)).reshape(1, HID)
    ls, o = _tc_final(agg2, t_hroot, b2p)
    return (ls[:, :NCLS], o[:, :NCLS])
